# baseline (speedup 1.0000x reference)
import functools

import jax
import jax.numpy as jnp
from jax import lax
from jax.experimental import pallas as pl
from jax.experimental.pallas import tpu as pltpu
from jax.experimental.pallas import tpu_sc as plsc

N = 10000
D = 128
E = 320000
CHUNK = 80
NC, NS = 2, 16
NW = NC * NS
EPT = E // NW
NCH = EPT // CHUNK
ROWS_PER_TILE = 624
TAIL_ROWS = N - NS * ROWS_PER_TILE
ZROWS = 16
NREL = 200
HALF = NREL // 2
ASYM_PAD = 224


def _sc_aggregate(x, src, dst, etype, alpha_flat):
    mesh = plsc.VectorSubcoreMesh(core_axis_name="c", subcore_axis_name="s")

    @functools.partial(
        pl.kernel,
        mesh=mesh,
        out_type=jax.ShapeDtypeStruct((NC, N, D), jnp.float32),
        scratch_types=[
            pltpu.VMEM((EPT,), jnp.int32),
            pltpu.VMEM((CHUNK,), jnp.int32),
            pltpu.VMEM((CHUNK,), jnp.int32),
            pltpu.VMEM((CHUNK,), jnp.int32),
            pltpu.VMEM((CHUNK,), jnp.int32),
            pltpu.VMEM((CHUNK,), jnp.int32),
            pltpu.VMEM((CHUNK,), jnp.int32),
            pltpu.VMEM((CHUNK, D), jnp.float32),
            pltpu.VMEM((CHUNK, D), jnp.float32),
            pltpu.VMEM((CHUNK, D), jnp.float32),
            pltpu.VMEM((NREL,), jnp.float32),
            pltpu.VMEM((ASYM_PAD,), jnp.float32),
            pltpu.VMEM((ZROWS, D), jnp.float32),
            pltpu.VMEM_SHARED((N, D), jnp.float32),
            pltpu.SemaphoreType.DMA,
            pltpu.SemaphoreType.DMA,
            pltpu.SemaphoreType.DMA,
            pltpu.SemaphoreType.DMA,
            pltpu.SemaphoreType.DMA,
        ],
    )
    def k(x_hbm, src_hbm, dst_hbm, type_hbm, alpha_hbm, out_hbm,
          srcs_v, type0, type1, type2, dst0, dst1, dst2, rows0, rows1, rows2,
          araw_v, asym_v, zero_v, acc_sh, gsem, dsem, ssem0, ssem1, ssem2):
        cid = lax.axis_index("c")
        sid = lax.axis_index("s")
        wid = sid * NC + cid

        buffers = ((rows0, dst0, type0, ssem0),
                   (rows1, dst1, type1, ssem1),
                   (rows2, dst2, type2, ssem2))

        ebase = wid * EPT
        pltpu.sync_copy(src_hbm.at[pl.ds(ebase, EPT)], srcs_v)

        def g_start(c, b):
            rows_b, dst_b, type_b, _ = buffers[b]
            pltpu.async_copy(x_hbm.at[srcs_v.at[pl.ds(c * CHUNK, CHUNK)]],
                             rows_b, gsem)
            pltpu.async_copy(dst_hbm.at[pl.ds(ebase + c * CHUNK, CHUNK)],
                             dst_b, dsem)
            pltpu.async_copy(type_hbm.at[pl.ds(ebase + c * CHUNK, CHUNK)],
                             type_b, dsem)

        def g_wait(b):
            rows_b, dst_b, type_b, _ = buffers[b]
            pltpu.make_async_copy(
                x_hbm.at[srcs_v.at[pl.ds(0, CHUNK)]], rows_b, gsem).wait()
            pltpu.make_async_copy(
                dst_hbm.at[pl.ds(0, CHUNK)], dst_b, dsem).wait()
            pltpu.make_async_copy(
                type_hbm.at[pl.ds(0, CHUNK)], type_b, dsem).wait()

        def s_start(b):
            rows_b, dst_b, _, ssem_b = buffers[b]
            pltpu.async_copy(rows_b, acc_sh.at[dst_b], ssem_b, add=True)

        def s_wait(b):
            rows_b, dst_b, _, ssem_b = buffers[b]
            pltpu.make_async_copy(rows_b, acc_sh.at[dst_b], ssem_b).wait()

        def scale(b):
            rows_b, _, type_b, _ = buffers[b]

            def grp(g, _):
                tv = type_b[pl.ds(g * 16, 16)]
                for k16 in range(16):
                    e = g * 16 + k16
                    av = asym_v[pl.ds(tv[k16], 16)]
                    a_spl = jnp.full((16,), av[0], jnp.float32)
                    for cg in range(8):
                        sl = pl.ds(cg * 16, 16)
                        rows_b[e, sl] = rows_b[e, sl] * a_spl
                return 0

            lax.fori_loop(0, CHUNK // 16, grp, 0)

        g_start(0, 0)

        pltpu.sync_copy(alpha_hbm, araw_v)
        lane = lax.broadcasted_iota(jnp.int32, (16,), 0)
        for r0 in (0, 16, 32, 48, 64, 80, 84):
            lo = araw_v[pl.ds(r0, 16)]
            hi = araw_v[pl.ds(r0 + HALF, 16)]
            if r0 == 0:
                lo = jnp.where(lane == 0, 0.0, lo)
            s = lo + hi
            asym_v[pl.ds(r0, 16)] = s
            asym_v[pl.ds(r0 + HALF, 16)] = s

        z16 = jnp.zeros((16,), jnp.float32)

        def zfill(i, _):
            zero_v[i // 8, pl.ds((i % 8) * 16, 16)] = z16
            return 0

        lax.fori_loop(0, ZROWS * 8, zfill, 0)
        base = sid * ROWS_PER_TILE

        def zcopy(i, _):
            pltpu.sync_copy(zero_v, acc_sh.at[pl.ds(base + i * ZROWS, ZROWS)])
            return 0

        lax.fori_loop(0, ROWS_PER_TILE // ZROWS, zcopy, 0)

        @pl.when(sid == NS - 1)
        def _zero_tail():
            pltpu.sync_copy(zero_v.at[pl.ds(0, TAIL_ROWS)],
                            acc_sh.at[pl.ds(NS * ROWS_PER_TILE, TAIL_ROWS)])

        plsc.subcore_barrier()

        def step(c, bcur, bnext, first, last):
            g_wait(bcur)
            if not last:
                if not first:
                    s_wait(bnext)
                g_start(c + 1, bnext)
            scale(bcur)
            s_start(bcur)

        step(0, 0, 1, True, False)
        step(1, 1, 2, True, False)

        def triple(i, _):
            c = 2 + 3 * i
            step(c, 2, 0, False, False)
            step(c + 1, 0, 1, False, False)
            step(c + 2, 1, 2, False, False)
            return 0

        lax.fori_loop(0, (NCH - 5) // 3, triple, 0)
        step(NCH - 3, 2, 0, False, False)
        step(NCH - 2, 0, 1, False, False)
        step(NCH - 1, 1, 2, False, True)

        s_wait(0)
        s_wait(1)
        s_wait(2)
        plsc.subcore_barrier()

        pltpu.sync_copy(acc_sh.at[pl.ds(base, ROWS_PER_TILE)],
                        out_hbm.at[cid, pl.ds(base, ROWS_PER_TILE)])

        @pl.when(sid == NS - 1)
        def _drain_tail():
            pltpu.sync_copy(acc_sh.at[pl.ds(NS * ROWS_PER_TILE, TAIL_ROWS)],
                            out_hbm.at[cid, pl.ds(NS * ROWS_PER_TILE, TAIL_ROWS)])

    return k(x, src, dst, etype, alpha_flat)


def _tc_finish(partials, W, gamma, beta):
    def body(p_ref, w_ref, g_ref, b_ref, o_ref):
        agg = p_ref[0] + p_ref[1]
        feats = jnp.dot(agg, w_ref[...], preferred_element_type=jnp.float32)
        mean = jnp.mean(feats, axis=0, keepdims=True)
        dd = feats - mean
        var = jnp.mean(dd * dd, axis=0, keepdims=True)
        o_ref[...] = dd * lax.rsqrt(var + 1e-5) * g_ref[...] + b_ref[...]

    return pl.pallas_call(
        body,
        out_shape=jax.ShapeDtypeStruct((N, D), jnp.float32),
    )(partials, W, gamma.reshape(1, D), beta.reshape(1, D))


def kernel(x, edge_index, all_edge_type, W, alpha_table, gamma, beta):
    partials = _sc_aggregate(x, edge_index[0], edge_index[1], all_edge_type,
                             alpha_table.reshape(NREL))
    return _tc_finish(partials, W, gamma, beta)


# speedup vs baseline: 1.0184x; 1.0184x over previous
import functools

import jax
import jax.numpy as jnp
from jax import lax
from jax.experimental import pallas as pl
from jax.experimental.pallas import tpu as pltpu
from jax.experimental.pallas import tpu_sc as plsc

N = 10000
D = 128
E = 320000
CHUNK = 80
NC, NS = 2, 16
NW = NC * NS
EPT = E // NW
NCH = EPT // CHUNK
ROWS_PER_TILE = 624
TAIL_ROWS = N - NS * ROWS_PER_TILE
ZROWS = 16
NREL = 200
HALF = NREL // 2
ASYM_PAD = 224


def _sc_aggregate(x, src, dst, etype, alpha_flat):
    mesh = plsc.VectorSubcoreMesh(core_axis_name="c", subcore_axis_name="s")

    @functools.partial(
        pl.kernel,
        mesh=mesh,
        compiler_params=pltpu.CompilerParams(needs_layout_passes=False),
        out_type=jax.ShapeDtypeStruct((NC, N, D), jnp.float32),
        scratch_types=[
            pltpu.VMEM((EPT,), jnp.int32),
            pltpu.VMEM((CHUNK,), jnp.int32),
            pltpu.VMEM((CHUNK,), jnp.int32),
            pltpu.VMEM((CHUNK,), jnp.int32),
            pltpu.VMEM((CHUNK,), jnp.int32),
            pltpu.VMEM((CHUNK,), jnp.int32),
            pltpu.VMEM((CHUNK,), jnp.int32),
            pltpu.VMEM((CHUNK, D), jnp.float32),
            pltpu.VMEM((CHUNK, D), jnp.float32),
            pltpu.VMEM((CHUNK, D), jnp.float32),
            pltpu.VMEM((NREL,), jnp.float32),
            pltpu.VMEM((ASYM_PAD,), jnp.float32),
            pltpu.VMEM((ZROWS, D), jnp.float32),
            pltpu.VMEM_SHARED((N, D), jnp.float32),
            pltpu.SemaphoreType.DMA,
            pltpu.SemaphoreType.DMA,
            pltpu.SemaphoreType.DMA,
            pltpu.SemaphoreType.DMA,
            pltpu.SemaphoreType.DMA,
        ],
    )
    def k(x_hbm, src_hbm, dst_hbm, type_hbm, alpha_hbm, out_hbm,
          srcs_v, type0, type1, type2, dst0, dst1, dst2, rows0, rows1, rows2,
          araw_v, asym_v, zero_v, acc_sh, gsem, dsem, ssem0, ssem1, ssem2):
        cid = lax.axis_index("c")
        sid = lax.axis_index("s")
        wid = sid * NC + cid

        buffers = ((rows0, dst0, type0, ssem0),
                   (rows1, dst1, type1, ssem1),
                   (rows2, dst2, type2, ssem2))

        ebase = wid * EPT
        pltpu.sync_copy(src_hbm.at[pl.ds(ebase, EPT)], srcs_v)

        def g_start(c, b):
            rows_b, dst_b, type_b, _ = buffers[b]
            pltpu.async_copy(x_hbm.at[srcs_v.at[pl.ds(c * CHUNK, CHUNK)]],
                             rows_b, gsem)
            pltpu.async_copy(dst_hbm.at[pl.ds(ebase + c * CHUNK, CHUNK)],
                             dst_b, dsem)
            pltpu.async_copy(type_hbm.at[pl.ds(ebase + c * CHUNK, CHUNK)],
                             type_b, dsem)

        def g_wait(b):
            rows_b, dst_b, type_b, _ = buffers[b]
            pltpu.make_async_copy(
                x_hbm.at[srcs_v.at[pl.ds(0, CHUNK)]], rows_b, gsem).wait()
            pltpu.make_async_copy(
                dst_hbm.at[pl.ds(0, CHUNK)], dst_b, dsem).wait()
            pltpu.make_async_copy(
                type_hbm.at[pl.ds(0, CHUNK)], type_b, dsem).wait()

        def s_start(b):
            rows_b, dst_b, _, ssem_b = buffers[b]
            pltpu.async_copy(rows_b, acc_sh.at[dst_b], ssem_b, add=True)

        def s_wait(b):
            rows_b, dst_b, _, ssem_b = buffers[b]
            pltpu.make_async_copy(rows_b, acc_sh.at[dst_b], ssem_b).wait()

        def scale(b):
            rows_b, _, type_b, _ = buffers[b]

            def grp(g, _):
                tv = type_b[pl.ds(g * 16, 16)]
                av = plsc.load_gather(asym_v, [tv])
                for k16 in range(16):
                    e = g * 16 + k16
                    a_spl = av[jnp.full((16,), k16, jnp.int32)]
                    for cg in range(8):
                        sl = pl.ds(cg * 16, 16)
                        rows_b[e, sl] = rows_b[e, sl] * a_spl
                return 0

            lax.fori_loop(0, CHUNK // 16, grp, 0)

        g_start(0, 0)

        pltpu.sync_copy(alpha_hbm, araw_v)
        lane = lax.broadcasted_iota(jnp.int32, (16,), 0)
        for r0 in (0, 16, 32, 48, 64, 80, 84):
            lo = araw_v[pl.ds(r0, 16)]
            hi = araw_v[pl.ds(r0 + HALF, 16)]
            if r0 == 0:
                lo = jnp.where(lane == 0, 0.0, lo)
            s = lo + hi
            asym_v[pl.ds(r0, 16)] = s
            asym_v[pl.ds(r0 + HALF, 16)] = s

        z16 = jnp.zeros((16,), jnp.float32)

        def zfill(i, _):
            zero_v[i // 8, pl.ds((i % 8) * 16, 16)] = z16
            return 0

        lax.fori_loop(0, ZROWS * 8, zfill, 0)
        base = sid * ROWS_PER_TILE

        def zcopy(i, _):
            pltpu.sync_copy(zero_v, acc_sh.at[pl.ds(base + i * ZROWS, ZROWS)])
            return 0

        lax.fori_loop(0, ROWS_PER_TILE // ZROWS, zcopy, 0)

        @pl.when(sid == NS - 1)
        def _zero_tail():
            pltpu.sync_copy(zero_v.at[pl.ds(0, TAIL_ROWS)],
                            acc_sh.at[pl.ds(NS * ROWS_PER_TILE, TAIL_ROWS)])

        plsc.subcore_barrier()

        def step(c, bcur, bnext, first, last):
            g_wait(bcur)
            if not last:
                if not first:
                    s_wait(bnext)
                g_start(c + 1, bnext)
            scale(bcur)
            s_start(bcur)

        step(0, 0, 1, True, False)
        step(1, 1, 2, True, False)

        def triple(i, _):
            c = 2 + 3 * i
            step(c, 2, 0, False, False)
            step(c + 1, 0, 1, False, False)
            step(c + 2, 1, 2, False, False)
            return 0

        lax.fori_loop(0, (NCH - 5) // 3, triple, 0)
        step(NCH - 3, 2, 0, False, False)
        step(NCH - 2, 0, 1, False, False)
        step(NCH - 1, 1, 2, False, True)

        s_wait(0)
        s_wait(1)
        s_wait(2)
        plsc.subcore_barrier()

        pltpu.sync_copy(acc_sh.at[pl.ds(base, ROWS_PER_TILE)],
                        out_hbm.at[cid, pl.ds(base, ROWS_PER_TILE)])

        @pl.when(sid == NS - 1)
        def _drain_tail():
            pltpu.sync_copy(acc_sh.at[pl.ds(NS * ROWS_PER_TILE, TAIL_ROWS)],
                            out_hbm.at[cid, pl.ds(NS * ROWS_PER_TILE, TAIL_ROWS)])

    return k(x, src, dst, etype, alpha_flat)


def _tc_finish(partials, W, gamma, beta):
    def body(p_ref, w_ref, g_ref, b_ref, o_ref):
        agg = p_ref[0] + p_ref[1]
        feats = jnp.dot(agg, w_ref[...], preferred_element_type=jnp.float32)
        mean = jnp.mean(feats, axis=0, keepdims=True)
        dd = feats - mean
        var = jnp.mean(dd * dd, axis=0, keepdims=True)
        o_ref[...] = dd * lax.rsqrt(var + 1e-5) * g_ref[...] + b_ref[...]

    return pl.pallas_call(
        body,
        out_shape=jax.ShapeDtypeStruct((N, D), jnp.float32),
    )(partials, W, gamma.reshape(1, D), beta.reshape(1, D))


def kernel(x, edge_index, all_edge_type, W, alpha_table, gamma, beta):
    partials = _sc_aggregate(x, edge_index[0], edge_index[1], all_edge_type,
                             alpha_table.reshape(NREL))
    return _tc_finish(partials, W, gamma, beta)


# speedup vs baseline: 1.0193x; 1.0009x over previous
import functools

import jax
import jax.numpy as jnp
from jax import lax
from jax.experimental import pallas as pl
from jax.experimental.pallas import tpu as pltpu
from jax.experimental.pallas import tpu_sc as plsc

N = 10000
D = 128
E = 320000
CHUNK = 80
NC, NS = 2, 16
NW = NC * NS
EPT = E // NW
NCH = EPT // CHUNK
ROWS_PER_TILE = 624
TAIL_ROWS = N - NS * ROWS_PER_TILE
ZROWS = 16
NREL = 200
HALF = NREL // 2
ASYM_PAD = 224


def _sc_aggregate(x, src, dst, etype, alpha_flat):
    mesh = plsc.VectorSubcoreMesh(core_axis_name="c", subcore_axis_name="s")

    @functools.partial(
        pl.kernel,
        mesh=mesh,
        compiler_params=pltpu.CompilerParams(needs_layout_passes=False),
        out_type=jax.ShapeDtypeStruct((NC, N, D), jnp.float32),
        scratch_types=[
            pltpu.VMEM((EPT,), jnp.int32),
            pltpu.VMEM((CHUNK,), jnp.int32),
            pltpu.VMEM((CHUNK,), jnp.int32),
            pltpu.VMEM((CHUNK,), jnp.int32),
            pltpu.VMEM((CHUNK,), jnp.int32),
            pltpu.VMEM((CHUNK,), jnp.int32),
            pltpu.VMEM((CHUNK,), jnp.int32),
            pltpu.VMEM((CHUNK, D), jnp.float32),
            pltpu.VMEM((CHUNK, D), jnp.float32),
            pltpu.VMEM((CHUNK, D), jnp.float32),
            pltpu.VMEM((NREL,), jnp.float32),
            pltpu.VMEM((ASYM_PAD,), jnp.float32),
            pltpu.VMEM((ZROWS, D), jnp.float32),
            pltpu.VMEM_SHARED((N, D), jnp.float32),
            pltpu.SemaphoreType.DMA,
            pltpu.SemaphoreType.DMA,
            pltpu.SemaphoreType.DMA,
            pltpu.SemaphoreType.DMA,
            pltpu.SemaphoreType.DMA,
        ],
    )
    def k(x_hbm, src_hbm, dst_hbm, type_hbm, alpha_hbm, out_hbm,
          srcs_v, type0, type1, type2, dst0, dst1, dst2, rows0, rows1, rows2,
          araw_v, asym_v, zero_v, acc_sh, gsem, dsem, ssem0, ssem1, ssem2):
        cid = lax.axis_index("c")
        sid = lax.axis_index("s")
        wid = sid * NC + cid

        buffers = ((rows0, dst0, type0, ssem0),
                   (rows1, dst1, type1, ssem1),
                   (rows2, dst2, type2, ssem2))

        ebase = wid * EPT
        pltpu.sync_copy(src_hbm.at[pl.ds(ebase, EPT)], srcs_v)

        HC = CHUNK // 2

        def g_start(c, b):
            rows_b, dst_b, type_b, _ = buffers[b]
            pltpu.async_copy(x_hbm.at[srcs_v.at[pl.ds(c * CHUNK, HC)]],
                             rows_b.at[pl.ds(0, HC)], gsem)
            pltpu.async_copy(x_hbm.at[srcs_v.at[pl.ds(c * CHUNK + HC, HC)]],
                             rows_b.at[pl.ds(HC, HC)], gsem)
            pltpu.async_copy(dst_hbm.at[pl.ds(ebase + c * CHUNK, CHUNK)],
                             dst_b, dsem)
            pltpu.async_copy(type_hbm.at[pl.ds(ebase + c * CHUNK, CHUNK)],
                             type_b, dsem)

        def g_wait(b):
            rows_b, dst_b, type_b, _ = buffers[b]
            pltpu.make_async_copy(
                x_hbm.at[srcs_v.at[pl.ds(0, HC)]],
                rows_b.at[pl.ds(0, HC)], gsem).wait()
            pltpu.make_async_copy(
                x_hbm.at[srcs_v.at[pl.ds(0, HC)]],
                rows_b.at[pl.ds(HC, HC)], gsem).wait()
            pltpu.make_async_copy(
                dst_hbm.at[pl.ds(0, CHUNK)], dst_b, dsem).wait()
            pltpu.make_async_copy(
                type_hbm.at[pl.ds(0, CHUNK)], type_b, dsem).wait()

        def s_start(b):
            rows_b, dst_b, _, ssem_b = buffers[b]
            pltpu.async_copy(rows_b, acc_sh.at[dst_b], ssem_b, add=True)

        def s_wait(b):
            rows_b, dst_b, _, ssem_b = buffers[b]
            pltpu.make_async_copy(rows_b, acc_sh.at[dst_b], ssem_b).wait()

        def scale(b):
            rows_b, _, type_b, _ = buffers[b]

            def grp(g, _):
                tv = type_b[pl.ds(g * 16, 16)]
                av = plsc.load_gather(asym_v, [tv])
                for k16 in range(16):
                    e = g * 16 + k16
                    a_spl = av[jnp.full((16,), k16, jnp.int32)]
                    for cg in range(8):
                        sl = pl.ds(cg * 16, 16)
                        rows_b[e, sl] = rows_b[e, sl] * a_spl
                return 0

            lax.fori_loop(0, CHUNK // 16, grp, 0)

        g_start(0, 0)

        pltpu.sync_copy(alpha_hbm, araw_v)
        lane = lax.broadcasted_iota(jnp.int32, (16,), 0)
        for r0 in (0, 16, 32, 48, 64, 80, 84):
            lo = araw_v[pl.ds(r0, 16)]
            hi = araw_v[pl.ds(r0 + HALF, 16)]
            if r0 == 0:
                lo = jnp.where(lane == 0, 0.0, lo)
            s = lo + hi
            asym_v[pl.ds(r0, 16)] = s
            asym_v[pl.ds(r0 + HALF, 16)] = s

        z16 = jnp.zeros((16,), jnp.float32)

        def zfill(i, _):
            zero_v[i // 8, pl.ds((i % 8) * 16, 16)] = z16
            return 0

        lax.fori_loop(0, ZROWS * 8, zfill, 0)
        base = sid * ROWS_PER_TILE

        def zcopy(i, _):
            pltpu.sync_copy(zero_v, acc_sh.at[pl.ds(base + i * ZROWS, ZROWS)])
            return 0

        lax.fori_loop(0, ROWS_PER_TILE // ZROWS, zcopy, 0)

        @pl.when(sid == NS - 1)
        def _zero_tail():
            pltpu.sync_copy(zero_v.at[pl.ds(0, TAIL_ROWS)],
                            acc_sh.at[pl.ds(NS * ROWS_PER_TILE, TAIL_ROWS)])

        plsc.subcore_barrier()

        def step(c, bcur, bnext, first, last):
            g_wait(bcur)
            if not last:
                if not first:
                    s_wait(bnext)
                g_start(c + 1, bnext)
            scale(bcur)
            s_start(bcur)

        step(0, 0, 1, True, False)
        step(1, 1, 2, True, False)

        def triple(i, _):
            c = 2 + 3 * i
            step(c, 2, 0, False, False)
            step(c + 1, 0, 1, False, False)
            step(c + 2, 1, 2, False, False)
            return 0

        lax.fori_loop(0, (NCH - 5) // 3, triple, 0)
        step(NCH - 3, 2, 0, False, False)
        step(NCH - 2, 0, 1, False, False)
        step(NCH - 1, 1, 2, False, True)

        s_wait(0)
        s_wait(1)
        s_wait(2)
        plsc.subcore_barrier()

        pltpu.sync_copy(acc_sh.at[pl.ds(base, ROWS_PER_TILE)],
                        out_hbm.at[cid, pl.ds(base, ROWS_PER_TILE)])

        @pl.when(sid == NS - 1)
        def _drain_tail():
            pltpu.sync_copy(acc_sh.at[pl.ds(NS * ROWS_PER_TILE, TAIL_ROWS)],
                            out_hbm.at[cid, pl.ds(NS * ROWS_PER_TILE, TAIL_ROWS)])

    return k(x, src, dst, etype, alpha_flat)


def _tc_finish(partials, W, gamma, beta):
    def body(p_ref, w_ref, g_ref, b_ref, o_ref):
        agg = p_ref[0] + p_ref[1]
        feats = jnp.dot(agg, w_ref[...], preferred_element_type=jnp.float32)
        mean = jnp.mean(feats, axis=0, keepdims=True)
        dd = feats - mean
        var = jnp.mean(dd * dd, axis=0, keepdims=True)
        o_ref[...] = dd * lax.rsqrt(var + 1e-5) * g_ref[...] + b_ref[...]

    return pl.pallas_call(
        body,
        out_shape=jax.ShapeDtypeStruct((N, D), jnp.float32),
    )(partials, W, gamma.reshape(1, D), beta.reshape(1, D))


def kernel(x, edge_index, all_edge_type, W, alpha_table, gamma, beta):
    partials = _sc_aggregate(x, edge_index[0], edge_index[1], all_edge_type,
                             alpha_table.reshape(NREL))
    return _tc_finish(partials, W, gamma, beta)
